# baseline (speedup 1.0000x reference)
import functools

import jax
import jax.numpy as jnp
from jax import lax
from jax.experimental import pallas as pl
from jax.experimental.pallas import tpu as pltpu
from jax.experimental.pallas import tpu_sc as plsc

D = 128
B = 4096
CHUNK = 128
NW = 32

_GATHERS = (
    ("cemb", 1),
    ("pemb", 1),
    ("patn", 32),
    ("ind", 4),
    ("fp", 2),
    ("ipc", 8),
    ("date", 1),
)


def _sc_gather_all(patent_table, patentee_table, ipc_table, industry_table,
                   appdate_table, idx_chunks):
    mesh = plsc.VectorSubcoreMesh(core_axis_name="c", subcore_axis_name="s")
    nch = {name: idx_chunks[name].shape[0] for name, _ in _GATHERS}
    out_types = [jax.ShapeDtypeStruct((nch[name], CHUNK, D), jnp.float32)
                 for name, _ in _GATHERS]

    @functools.partial(
        pl.kernel, mesh=mesh,
        out_type=out_types,
        scratch_types=[
            pltpu.VMEM((CHUNK,), jnp.int32),
            pltpu.VMEM((CHUNK, D), jnp.float32),
            pltpu.SemaphoreType.DMA,
        ],
    )
    def k(pat_t, pee_t, ipc_t, ind_t, date_t,
          cemb_i, pemb_i, patn_i, ind_i, fp_i, ipc_i, date_i,
          cemb_o, pemb_o, patn_o, ind_o, fp_o, ipc_o, date_o,
          idx_v, rows_v, sem):
        wid = lax.axis_index("s") * 2 + lax.axis_index("c")

        def do_gather(table_ref, idx_ref, out_ref, nchunks):
            per_w = nchunks // NW
            base = wid * per_w

            def body(i, carry):
                c = base + i
                pltpu.sync_copy(idx_ref.at[c], idx_v)
                pltpu.async_copy(table_ref.at[idx_v], rows_v, sem).wait()
                pltpu.sync_copy(rows_v, out_ref.at[c])
                return carry

            lax.fori_loop(0, per_w, body, 0)

        do_gather(pee_t, cemb_i, cemb_o, nch["cemb"])
        do_gather(pat_t, pemb_i, pemb_o, nch["pemb"])
        do_gather(pat_t, patn_i, patn_o, nch["patn"])
        do_gather(ind_t, ind_i, ind_o, nch["ind"])
        do_gather(pee_t, fp_i, fp_o, nch["fp"])
        do_gather(ipc_t, ipc_i, ipc_o, nch["ipc"])
        do_gather(date_t, date_i, date_o, nch["date"])

    outs = k(patent_table, patentee_table, ipc_table, industry_table,
             appdate_table, *[idx_chunks[name] for name, _ in _GATHERS])
    return {name: o for (name, _), o in zip(_GATHERS, outs)}


def _compute_body(cemb_ref, pemb_ref, patn_ref, ind_ref, fp_ref, ipc_ref,
                  date_ref, wagg_ref, bagg_ref, wfil_ref, bfil_ref, out_ref):
    wa1 = wagg_ref[:D, :]
    wa2 = wagg_ref[D:, :]
    wf1 = wfil_ref[:D, :]
    wf2 = wfil_ref[D:, :]
    bagg = bagg_ref[...]
    bfil = bfil_ref[...]

    bf = jnp.bfloat16
    wa1h, wa2h, wf1h, wf2h = (w.astype(bf) for w in (wa1, wa2, wf1, wf2))

    def side(center, groups, n_total):
        bb = center.shape[0]
        c_w = jnp.dot(center.astype(bf), wa1h, preferred_element_type=jnp.float32)
        gsum = jnp.zeros((bb, D), jnp.float32)
        ssum = jnp.zeros((bb, D), jnp.float32)
        for rows, n in groups:
            h = jnp.dot(rows.astype(bf), wa2h, preferred_element_type=jnp.float32)
            h3 = h.reshape(bb, n, D)
            r3 = rows.reshape(bb, n, D)
            gate = jax.nn.sigmoid(h3 + c_w[:, None, :] + bagg[None])
            gsum = gsum + jnp.sum(r3 * gate, axis=1)
            ssum = ssum + jnp.sum(r3, axis=1)
        agg = gsum * (1.0 / n_total)
        avg = ssum * (1.0 / n_total)
        fg = jax.nn.sigmoid(
            jnp.dot(center.astype(bf), wf1h, preferred_element_type=jnp.float32)
            + jnp.dot(avg.astype(bf), wf2h, preferred_element_type=jnp.float32)
            + bfil)
        x = center * (1.0 - fg) + agg
        return jnp.where(x >= 0, x, 0.2 * x)

    cemb = cemb_ref[...]
    pemb = pemb_ref[...]
    cs = side(cemb, [(ind_ref[...], 4), (patn_ref[...], 32)], 36.0)
    ps = side(pemb, [(fp_ref[...], 2), (ipc_ref[...], 8), (date_ref[...], 1)],
              11.0)
    out_ref[...] = jax.nn.sigmoid(jnp.sum(cs * ps, axis=1, keepdims=True))


def _tc_compute(cemb, pemb, patn, ind, fp, ipc, date, W_agg, b_agg, W_fil,
                b_fil, bb=256, interpret=False):
    nblk = B // bb

    def row_spec(n):
        return pl.BlockSpec((bb * n, D), lambda i: (i, 0))

    full = lambda shape: pl.BlockSpec(shape, lambda i: (0, 0))
    out = pl.pallas_call(
        _compute_body,
        grid=(nblk,),
        in_specs=[
            row_spec(1), row_spec(1), row_spec(32), row_spec(4), row_spec(2),
            row_spec(8), row_spec(1),
            full((2 * D, D)), full((1, D)), full((2 * D, D)), full((1, D)),
        ],
        out_specs=pl.BlockSpec((bb, 1), lambda i: (i, 0)),
        out_shape=jax.ShapeDtypeStruct((B, 1), jnp.float32),
        interpret=interpret,
    )(cemb, pemb, patn, ind, fp, ipc, date, W_agg, b_agg.reshape(1, D),
      W_fil, b_fil.reshape(1, D))
    return out.reshape(B)


def kernel(company_ids, patent_ids, patent_neighbors, industry_neighbors,
           first_patentee_neighbors, ipc_neighbors, appdate_neighbors,
           patent_table, patentee_table, ipc_table, industry_table,
           appdate_table, W_agg, b_agg, W_fil, b_fil):
    idx_flat = {
        "cemb": company_ids,
        "pemb": patent_ids,
        "patn": patent_neighbors,
        "ind": industry_neighbors,
        "fp": first_patentee_neighbors,
        "ipc": ipc_neighbors,
        "date": appdate_neighbors,
    }
    idx_chunks = {
        name: a.astype(jnp.int32).reshape(-1, CHUNK)
        for name, a in idx_flat.items()
    }
    g = _sc_gather_all(patent_table, patentee_table, ipc_table,
                       industry_table, appdate_table, idx_chunks)
    rows = {name: v.reshape(-1, D) for name, v in g.items()}
    return _tc_compute(rows["cemb"], rows["pemb"], rows["patn"], rows["ind"],
                       rows["fp"], rows["ipc"], rows["date"],
                       W_agg, b_agg, W_fil, b_fil)


# speedup vs baseline: 1.1211x; 1.1211x over previous
import functools

import jax
import jax.numpy as jnp
from jax import lax
from jax.experimental import pallas as pl
from jax.experimental.pallas import tpu as pltpu
from jax.experimental.pallas import tpu_sc as plsc

D = 128
B = 4096
CHUNK = 128
NW = 32

_GATHERS = (
    ("cemb", 1),
    ("pemb", 1),
    ("patn", 32),
    ("ind", 4),
    ("fp", 2),
    ("ipc", 8),
    ("date", 1),
)


def _sc_gather_all(patent_table, patentee_table, ipc_table, industry_table,
                   appdate_table, idx_chunks):
    mesh = plsc.VectorSubcoreMesh(core_axis_name="c", subcore_axis_name="s")
    nch = {name: idx_chunks[name].shape[0] for name, _ in _GATHERS}
    out_types = [jax.ShapeDtypeStruct((nch[name], CHUNK, D), jnp.float32)
                 for name, _ in _GATHERS]

    @functools.partial(
        pl.kernel, mesh=mesh,
        out_type=out_types,
        scratch_types=[
            pltpu.VMEM((CHUNK,), jnp.int32),
            pltpu.VMEM((CHUNK, D), jnp.float32),
            pltpu.SemaphoreType.DMA,
        ],
    )
    def k(pat_t, pee_t, ipc_t, ind_t, date_t,
          cemb_i, pemb_i, patn_i, ind_i, fp_i, ipc_i, date_i,
          cemb_o, pemb_o, patn_o, ind_o, fp_o, ipc_o, date_o,
          idx_v, rows_v, sem):
        wid = lax.axis_index("s") * 2 + lax.axis_index("c")

        def do_gather(table_ref, idx_ref, out_ref, nchunks):
            trips = (nchunks - wid + NW - 1) // NW

            def body(i, carry):
                c = wid + i * NW
                pltpu.sync_copy(idx_ref.at[c], idx_v)
                pltpu.async_copy(table_ref.at[idx_v], rows_v, sem).wait()
                pltpu.sync_copy(rows_v, out_ref.at[c])
                return carry

            lax.fori_loop(0, trips, body, 0)

        do_gather(pee_t, cemb_i, cemb_o, nch["cemb"])
        do_gather(pat_t, pemb_i, pemb_o, nch["pemb"])
        do_gather(pat_t, patn_i, patn_o, nch["patn"])
        do_gather(ind_t, ind_i, ind_o, nch["ind"])
        do_gather(pee_t, fp_i, fp_o, nch["fp"])
        do_gather(ipc_t, ipc_i, ipc_o, nch["ipc"])
        do_gather(date_t, date_i, date_o, nch["date"])

    outs = k(patent_table, patentee_table, ipc_table, industry_table,
             appdate_table, *[idx_chunks[name] for name, _ in _GATHERS])
    return {name: o for (name, _), o in zip(_GATHERS, outs)}


def _compute_body(cemb_ref, pemb_ref, patn_ref, ind_ref, fp_ref, ipc_ref,
                  date_ref, wagg_ref, bagg_ref, wfil_ref, bfil_ref, out_ref):
    wa1 = wagg_ref[:D, :]
    wa2 = wagg_ref[D:, :]
    wf1 = wfil_ref[:D, :]
    wf2 = wfil_ref[D:, :]
    bagg = bagg_ref[...]
    bfil = bfil_ref[...]

    def side(center, groups, n_total):
        bb = center.shape[0]
        c_w = jnp.dot(center, wa1, preferred_element_type=jnp.float32)
        gsum = jnp.zeros((bb, D), jnp.float32)
        ssum = jnp.zeros((bb, D), jnp.float32)
        for rows, n in groups:
            h = jnp.dot(rows, wa2, preferred_element_type=jnp.float32)
            h3 = h.reshape(bb, n, D)
            r3 = rows.reshape(bb, n, D)
            gate = jax.nn.sigmoid(h3 + c_w[:, None, :] + bagg[None])
            gsum = gsum + jnp.sum(r3 * gate, axis=1)
            ssum = ssum + jnp.sum(r3, axis=1)
        agg = gsum * (1.0 / n_total)
        avg = ssum * (1.0 / n_total)
        fg = jax.nn.sigmoid(
            jnp.dot(center, wf1, preferred_element_type=jnp.float32)
            + jnp.dot(avg, wf2, preferred_element_type=jnp.float32) + bfil)
        x = center * (1.0 - fg) + agg
        return jnp.where(x >= 0, x, 0.2 * x)

    cemb = cemb_ref[...]
    pemb = pemb_ref[...]
    cs = side(cemb, [(ind_ref[...], 4), (patn_ref[...], 32)], 36.0)
    ps = side(pemb, [(fp_ref[...], 2), (ipc_ref[...], 8), (date_ref[...], 1)],
              11.0)
    out_ref[...] = jax.nn.sigmoid(jnp.sum(cs * ps, axis=1, keepdims=True))


def _tc_compute(cemb, pemb, patn, ind, fp, ipc, date, W_agg, b_agg, W_fil,
                b_fil, bb=256, interpret=False, nrows=B):
    nblk = nrows // bb

    def row_spec(n):
        return pl.BlockSpec((bb * n, D), lambda i: (i, 0))

    full = lambda shape: pl.BlockSpec(shape, lambda i: (0, 0))
    out = pl.pallas_call(
        _compute_body,
        grid=(nblk,),
        in_specs=[
            row_spec(1), row_spec(1), row_spec(32), row_spec(4), row_spec(2),
            row_spec(8), row_spec(1),
            full((2 * D, D)), full((1, D)), full((2 * D, D)), full((1, D)),
        ],
        out_specs=pl.BlockSpec((bb, 1), lambda i: (i, 0)),
        out_shape=jax.ShapeDtypeStruct((nrows, 1), jnp.float32),
        interpret=interpret,
    )(cemb, pemb, patn, ind, fp, ipc, date, W_agg, b_agg.reshape(1, D),
      W_fil, b_fil.reshape(1, D))
    return out.reshape(nrows)


def kernel(company_ids, patent_ids, patent_neighbors, industry_neighbors,
           first_patentee_neighbors, ipc_neighbors, appdate_neighbors,
           patent_table, patentee_table, ipc_table, industry_table,
           appdate_table, W_agg, b_agg, W_fil, b_fil):
    idx_flat = {
        "cemb": company_ids,
        "pemb": patent_ids,
        "patn": patent_neighbors,
        "ind": industry_neighbors,
        "fp": first_patentee_neighbors,
        "ipc": ipc_neighbors,
        "date": appdate_neighbors,
    }
    idx_chunks = {
        name: a.astype(jnp.int32).reshape(-1, CHUNK)
        for name, a in idx_flat.items()
    }
    S = 2
    bs = B // S
    gathered = []
    for s in range(S):
        sl = {name: lax.slice_in_dim(a, s * (a.shape[0] // S),
                                     (s + 1) * (a.shape[0] // S), axis=0)
              for name, a in idx_chunks.items()}
        gathered.append(_sc_gather_all(patent_table, patentee_table,
                                       ipc_table, industry_table,
                                       appdate_table, sl))
    outs = []
    for s in range(S):
        rows = {name: v.reshape(-1, D) for name, v in gathered[s].items()}
        outs.append(_tc_compute(rows["cemb"], rows["pemb"], rows["patn"],
                                rows["ind"], rows["fp"], rows["ipc"],
                                rows["date"], W_agg, b_agg, W_fil, b_fil,
                                nrows=bs))
    return jnp.concatenate(outs, axis=0)


# speedup vs baseline: 1.4301x; 1.2756x over previous
import functools

import jax
import jax.numpy as jnp
from jax import lax
from jax.experimental import pallas as pl
from jax.experimental.pallas import tpu as pltpu
from jax.experimental.pallas import tpu_sc as plsc

D = 128
B = 4096
CHUNK = 128
NW = 32

_GATHERS = (
    ("cemb", 1),
    ("pemb", 1),
    ("patn", 32),
    ("ind", 4),
    ("fp", 2),
    ("ipc", 8),
    ("date", 1),
)


def _sc_gather_all(patent_table, patentee_table, ipc_table, industry_table,
                   appdate_table, idx_flat):
    mesh = plsc.VectorSubcoreMesh(core_axis_name="c", subcore_axis_name="s")
    names = [name for name, _ in _GATHERS]
    R = {name: idx_flat[name].shape[0] for name in names}
    per_w = {name: R[name] // NW for name in names}
    idx_off = {}
    off = 0
    for name in names:
        idx_off[name] = off
        off += per_w[name]
    idx_total = off
    out_types = [jax.ShapeDtypeStruct((R[name], D), jnp.float32)
                 for name in names]

    @functools.partial(
        pl.kernel, mesh=mesh,
        out_type=out_types,
        scratch_types=[
            pltpu.VMEM((idx_total,), jnp.int32),
            pltpu.VMEM((CHUNK, D), jnp.float32),
            pltpu.VMEM((CHUNK, D), jnp.float32),
            pltpu.SemaphoreType.DMA,
            pltpu.SemaphoreType.DMA,
            pltpu.SemaphoreType.DMA,
            pltpu.SemaphoreType.DMA,
            pltpu.SemaphoreType.DMA,
        ],
    )
    def k(pat_t, pee_t, ipc_t, ind_t, date_t,
          cemb_i, pemb_i, patn_i, ind_i, fp_i, ipc_i, date_i,
          cemb_o, pemb_o, patn_o, ind_o, fp_o, ipc_o, date_o,
          idx_v, rows0, rows1, isem, gsem0, gsem1, wsem0, wsem1):
        wid = lax.axis_index("s") * 2 + lax.axis_index("c")
        tables = {"cemb": pee_t, "pemb": pat_t, "patn": pat_t, "ind": ind_t,
                  "fp": pee_t, "ipc": ipc_t, "date": date_t}
        idx_in = {"cemb": cemb_i, "pemb": pemb_i, "patn": patn_i,
                  "ind": ind_i, "fp": fp_i, "ipc": ipc_i, "date": date_i}
        outs = {"cemb": cemb_o, "pemb": pemb_o, "patn": patn_o, "ind": ind_o,
                "fp": fp_o, "ipc": ipc_o, "date": date_o}

        pre = []
        for name in names:
            pw = per_w[name]
            pre.append(pltpu.async_copy(
                idx_in[name].at[pl.ds(wid * pw, pw)],
                idx_v.at[pl.ds(idx_off[name], pw)], isem))
        for p in pre:
            p.wait()

        jobs = []
        for name in names:
            pw = per_w[name]
            ch = min(pw, CHUNK)
            for j in range(pw // ch):
                jobs.append((name, j, ch))
        jobs.sort(key=lambda t: -t[2])

        rows = (rows0, rows1)
        gsems = (gsem0, gsem1)
        wsems = (wsem0, wsem1)

        def fire_gather(job, s):
            name, j, ch = job
            src = tables[name].at[
                idx_v.at[pl.ds(idx_off[name] + j * ch, ch)]]
            dst = rows[s] if ch == CHUNK else rows[s].at[pl.ds(0, ch)]
            return pltpu.async_copy(src, dst, gsems[s])

        def fire_wb(job, s):
            name, j, ch = job
            src = rows[s] if ch == CHUNK else rows[s].at[pl.ds(0, ch)]
            dst = outs[name].at[pl.ds(wid * per_w[name] + j * ch, ch)]
            return pltpu.async_copy(src, dst, wsems[s])

        pend_w = [None, None]
        prev = None
        for n, job in enumerate(jobs):
            s = n % 2
            if pend_w[s] is not None:
                pend_w[s].wait()
                pend_w[s] = None
            g = fire_gather(job, s)
            if prev is not None:
                pg, pjob, ps = prev
                pg.wait()
                pend_w[ps] = fire_wb(pjob, ps)
            prev = (g, job, s)
        pg, pjob, ps = prev
        pg.wait()
        pend_w[ps] = fire_wb(pjob, ps)
        for s in (0, 1):
            if pend_w[s] is not None:
                pend_w[s].wait()

    outs = k(patent_table, patentee_table, ipc_table, industry_table,
             appdate_table, *[idx_flat[name] for name in names])
    return {name: o for name, o in zip(names, outs)}


def _compute_body(cemb_ref, pemb_ref, patn_ref, ind_ref, fp_ref, ipc_ref,
                  date_ref, wagg_ref, bagg_ref, wfil_ref, bfil_ref, out_ref):
    wa1 = wagg_ref[:D, :]
    wa2 = wagg_ref[D:, :]
    wf1 = wfil_ref[:D, :]
    wf2 = wfil_ref[D:, :]
    bagg = bagg_ref[...]
    bfil = bfil_ref[...]

    def side(center, groups, n_total):
        bb = center.shape[0]
        c_w = jnp.dot(center, wa1, preferred_element_type=jnp.float32)
        gsum = jnp.zeros((bb, D), jnp.float32)
        ssum = jnp.zeros((bb, D), jnp.float32)
        for rows, n in groups:
            h = jnp.dot(rows, wa2, preferred_element_type=jnp.float32)
            h3 = h.reshape(bb, n, D)
            r3 = rows.reshape(bb, n, D)
            gate = jax.nn.sigmoid(h3 + c_w[:, None, :] + bagg[None])
            gsum = gsum + jnp.sum(r3 * gate, axis=1)
            ssum = ssum + jnp.sum(r3, axis=1)
        agg = gsum * (1.0 / n_total)
        avg = ssum * (1.0 / n_total)
        fg = jax.nn.sigmoid(
            jnp.dot(center, wf1, preferred_element_type=jnp.float32)
            + jnp.dot(avg, wf2, preferred_element_type=jnp.float32) + bfil)
        x = center * (1.0 - fg) + agg
        return jnp.where(x >= 0, x, 0.2 * x)

    cemb = cemb_ref[...]
    pemb = pemb_ref[...]
    cs = side(cemb, [(ind_ref[...], 4), (patn_ref[...], 32)], 36.0)
    ps = side(pemb, [(fp_ref[...], 2), (ipc_ref[...], 8), (date_ref[...], 1)],
              11.0)
    out_ref[...] = jax.nn.sigmoid(jnp.sum(cs * ps, axis=1, keepdims=True))


def _tc_compute(cemb, pemb, patn, ind, fp, ipc, date, W_agg, b_agg, W_fil,
                b_fil, bb=256, interpret=False, nrows=B):
    nblk = nrows // bb

    def row_spec(n):
        return pl.BlockSpec((bb * n, D), lambda i: (i, 0))

    full = lambda shape: pl.BlockSpec(shape, lambda i: (0, 0))
    out = pl.pallas_call(
        _compute_body,
        grid=(nblk,),
        in_specs=[
            row_spec(1), row_spec(1), row_spec(32), row_spec(4), row_spec(2),
            row_spec(8), row_spec(1),
            full((2 * D, D)), full((1, D)), full((2 * D, D)), full((1, D)),
        ],
        out_specs=pl.BlockSpec((bb, 1), lambda i: (i, 0)),
        out_shape=jax.ShapeDtypeStruct((nrows, 1), jnp.float32),
        interpret=interpret,
    )(cemb, pemb, patn, ind, fp, ipc, date, W_agg, b_agg.reshape(1, D),
      W_fil, b_fil.reshape(1, D))
    return out.reshape(nrows)


def kernel(company_ids, patent_ids, patent_neighbors, industry_neighbors,
           first_patentee_neighbors, ipc_neighbors, appdate_neighbors,
           patent_table, patentee_table, ipc_table, industry_table,
           appdate_table, W_agg, b_agg, W_fil, b_fil):
    idx_flat = {
        "cemb": company_ids,
        "pemb": patent_ids,
        "patn": patent_neighbors,
        "ind": industry_neighbors,
        "fp": first_patentee_neighbors,
        "ipc": ipc_neighbors,
        "date": appdate_neighbors,
    }
    idx1d = {name: a.astype(jnp.int32).reshape(-1)
             for name, a in idx_flat.items()}
    S = 2
    bs = B // S
    gathered = []
    for s in range(S):
        sl = {name: lax.slice_in_dim(a, s * (a.shape[0] // S),
                                     (s + 1) * (a.shape[0] // S), axis=0)
              for name, a in idx1d.items()}
        gathered.append(_sc_gather_all(patent_table, patentee_table,
                                       ipc_table, industry_table,
                                       appdate_table, sl))
    outs = []
    for s in range(S):
        rows = gathered[s]
        outs.append(_tc_compute(rows["cemb"], rows["pemb"], rows["patn"],
                                rows["ind"], rows["fp"], rows["ipc"],
                                rows["date"], W_agg, b_agg, W_fil, b_fil,
                                nrows=bs))
    return jnp.concatenate(outs, axis=0)


# speedup vs baseline: 1.5954x; 1.1156x over previous
import functools

import jax
import jax.numpy as jnp
from jax import lax
from jax.experimental import pallas as pl
from jax.experimental.pallas import tpu as pltpu
from jax.experimental.pallas import tpu_sc as plsc

D = 128
B = 4096
CHUNK = 128
NW = 32
TCBB = 256

_GATHERS = (
    ("cemb", 1),
    ("pemb", 1),
    ("patn", 32),
    ("ind", 4),
    ("fp", 2),
    ("ipc", 8),
    ("date", 1),
)


def _sc_gather_all(patent_table, patentee_table, ipc_table, industry_table,
                   appdate_table, idx_flat):
    mesh = plsc.VectorSubcoreMesh(core_axis_name="c", subcore_axis_name="s")
    names = [name for name, _ in _GATHERS]
    R = {name: idx_flat[name].shape[0] for name in names}
    per_w = {name: R[name] // NW for name in names}
    idx_off = {}
    off = 0
    for name in names:
        idx_off[name] = off
        off += per_w[name]
    idx_total = off
    out_types = [jax.ShapeDtypeStruct((R[name], D), jnp.float32)
                 for name in names]

    @functools.partial(
        pl.kernel, mesh=mesh,
        out_type=out_types,
        scratch_types=[
            pltpu.VMEM((idx_total,), jnp.int32),
            pltpu.VMEM((CHUNK, D), jnp.float32),
            pltpu.VMEM((CHUNK, D), jnp.float32),
            pltpu.SemaphoreType.DMA,
            pltpu.SemaphoreType.DMA,
            pltpu.SemaphoreType.DMA,
            pltpu.SemaphoreType.DMA,
            pltpu.SemaphoreType.DMA,
        ],
    )
    def k(pat_t, pee_t, ipc_t, ind_t, date_t,
          cemb_i, pemb_i, patn_i, ind_i, fp_i, ipc_i, date_i,
          cemb_o, pemb_o, patn_o, ind_o, fp_o, ipc_o, date_o,
          idx_v, rows0, rows1, isem, gsem0, gsem1, wsem0, wsem1):
        wid = lax.axis_index("s") * 2 + lax.axis_index("c")
        tables = {"cemb": pee_t, "pemb": pat_t, "patn": pat_t, "ind": ind_t,
                  "fp": pee_t, "ipc": ipc_t, "date": date_t}
        idx_in = {"cemb": cemb_i, "pemb": pemb_i, "patn": patn_i,
                  "ind": ind_i, "fp": fp_i, "ipc": ipc_i, "date": date_i}
        outs = {"cemb": cemb_o, "pemb": pemb_o, "patn": patn_o, "ind": ind_o,
                "fp": fp_o, "ipc": ipc_o, "date": date_o}

        pre = []
        for name in names:
            pw = per_w[name]
            pre.append(pltpu.async_copy(
                idx_in[name].at[pl.ds(wid * pw, pw)],
                idx_v.at[pl.ds(idx_off[name], pw)], isem))
        for p in pre:
            p.wait()

        jobs = []
        for name in names:
            pw = per_w[name]
            ch = min(pw, CHUNK)
            for j in range(pw // ch):
                jobs.append((name, j, ch))
        jobs.sort(key=lambda t: -t[2])

        rows = (rows0, rows1)
        gsems = (gsem0, gsem1)
        wsems = (wsem0, wsem1)

        def fire_gather(job, s):
            name, j, ch = job
            src = tables[name].at[
                idx_v.at[pl.ds(idx_off[name] + j * ch, ch)]]
            dst = rows[s] if ch == CHUNK else rows[s].at[pl.ds(0, ch)]
            return pltpu.async_copy(src, dst, gsems[s])

        def fire_wb(job, s):
            name, j, ch = job
            src = rows[s] if ch == CHUNK else rows[s].at[pl.ds(0, ch)]
            dst = outs[name].at[pl.ds(wid * per_w[name] + j * ch, ch)]
            return pltpu.async_copy(src, dst, wsems[s])

        pend_w = [None, None]
        prev = None
        for n, job in enumerate(jobs):
            s = n % 2
            if pend_w[s] is not None:
                pend_w[s].wait()
                pend_w[s] = None
            g = fire_gather(job, s)
            if prev is not None:
                pg, pjob, ps = prev
                pg.wait()
                pend_w[ps] = fire_wb(pjob, ps)
            prev = (g, job, s)
        pg, pjob, ps = prev
        pg.wait()
        pend_w[ps] = fire_wb(pjob, ps)
        for s in (0, 1):
            if pend_w[s] is not None:
                pend_w[s].wait()

    outs = k(patent_table, patentee_table, ipc_table, industry_table,
             appdate_table, *[idx_flat[name] for name in names])
    return {name: o for name, o in zip(names, outs)}


def _compute_body(cemb_ref, pemb_ref, patn_ref, ind_ref, fp_ref, ipc_ref,
                  date_ref, wagg_ref, bagg_ref, wfil_ref, bfil_ref, out_ref):
    wa1 = wagg_ref[:D, :]
    wa2 = wagg_ref[D:, :]
    wf1 = wfil_ref[:D, :]
    wf2 = wfil_ref[D:, :]
    bagg = bagg_ref[...]
    bfil = bfil_ref[...]

    def side(center, groups, n_total):
        bb = center.shape[0]
        c_w = jnp.dot(center, wa1, preferred_element_type=jnp.float32) + bagg
        gsum = jnp.zeros((bb, D), jnp.float32)
        ssum = jnp.zeros((bb, D), jnp.float32)
        for rows, n in groups:
            h = jnp.dot(rows, wa2, preferred_element_type=jnp.float32)
            for k in range(n):
                r = rows[k * bb:(k + 1) * bb, :]
                gate = jax.nn.sigmoid(h[k * bb:(k + 1) * bb, :] + c_w)
                gsum = gsum + r * gate
                ssum = ssum + r
        agg = gsum * (1.0 / n_total)
        avg = ssum * (1.0 / n_total)
        fg = jax.nn.sigmoid(
            jnp.dot(center, wf1, preferred_element_type=jnp.float32)
            + jnp.dot(avg, wf2, preferred_element_type=jnp.float32) + bfil)
        x = center * (1.0 - fg) + agg
        return jnp.where(x >= 0, x, 0.2 * x)

    cemb = cemb_ref[...]
    pemb = pemb_ref[...]
    cs = side(cemb, [(ind_ref[...], 4), (patn_ref[...], 32)], 36.0)
    ps = side(pemb, [(fp_ref[...], 2), (ipc_ref[...], 8), (date_ref[...], 1)],
              11.0)
    out_ref[...] = jax.nn.sigmoid(jnp.sum(cs * ps, axis=1, keepdims=True))


def _tc_compute(cemb, pemb, patn, ind, fp, ipc, date, W_agg, b_agg, W_fil,
                b_fil, bb=TCBB, interpret=False, nrows=B):
    nblk = nrows // bb

    def row_spec(n):
        return pl.BlockSpec((bb * n, D), lambda i: (i, 0))

    full = lambda shape: pl.BlockSpec(shape, lambda i: (0, 0))
    out = pl.pallas_call(
        _compute_body,
        grid=(nblk,),
        in_specs=[
            row_spec(1), row_spec(1), row_spec(32), row_spec(4), row_spec(2),
            row_spec(8), row_spec(1),
            full((2 * D, D)), full((1, D)), full((2 * D, D)), full((1, D)),
        ],
        out_specs=pl.BlockSpec((bb, 1), lambda i: (i, 0)),
        out_shape=jax.ShapeDtypeStruct((nrows, 1), jnp.float32),
        interpret=interpret,
    )(cemb, pemb, patn, ind, fp, ipc, date, W_agg, b_agg.reshape(1, D),
      W_fil, b_fil.reshape(1, D))
    return out.reshape(nrows)


def kernel(company_ids, patent_ids, patent_neighbors, industry_neighbors,
           first_patentee_neighbors, ipc_neighbors, appdate_neighbors,
           patent_table, patentee_table, ipc_table, industry_table,
           appdate_table, W_agg, b_agg, W_fil, b_fil):
    idx_flat = {
        "cemb": company_ids,
        "pemb": patent_ids,
        "patn": patent_neighbors,
        "ind": industry_neighbors,
        "fp": first_patentee_neighbors,
        "ipc": ipc_neighbors,
        "date": appdate_neighbors,
    }
    def perm(a):
        a = a.astype(jnp.int32)
        n = 1 if a.ndim == 1 else a.shape[1]
        return (a.reshape(B // TCBB, TCBB, n).transpose(0, 2, 1).reshape(-1)
                if n > 1 else a.reshape(-1))

    idx1d = {name: perm(a) for name, a in idx_flat.items()}
    S = 2
    bs = B // S
    gathered = []
    for s in range(S):
        sl = {name: lax.slice_in_dim(a, s * (a.shape[0] // S),
                                     (s + 1) * (a.shape[0] // S), axis=0)
              for name, a in idx1d.items()}
        gathered.append(_sc_gather_all(patent_table, patentee_table,
                                       ipc_table, industry_table,
                                       appdate_table, sl))
    outs = []
    for s in range(S):
        rows = gathered[s]
        outs.append(_tc_compute(rows["cemb"], rows["pemb"], rows["patn"],
                                rows["ind"], rows["fp"], rows["ipc"],
                                rows["date"], W_agg, b_agg, W_fil, b_fil,
                                nrows=bs))
    return jnp.concatenate(outs, axis=0)


# speedup vs baseline: 1.5980x; 1.0016x over previous
import functools

import jax
import jax.numpy as jnp
from jax import lax
from jax.experimental import pallas as pl
from jax.experimental.pallas import tpu as pltpu
from jax.experimental.pallas import tpu_sc as plsc

D = 128
B = 4096
CHUNK = 128
NW = 32
TCBB = 256

_GATHERS = (
    ("cemb", 1),
    ("pemb", 1),
    ("patn", 32),
    ("ind", 4),
    ("fp", 2),
    ("ipc", 8),
    ("date", 1),
)


def _sc_gather_all(patent_table, patentee_table, ipc_table, industry_table,
                   appdate_table, idx_flat):
    mesh = plsc.VectorSubcoreMesh(core_axis_name="c", subcore_axis_name="s")
    names = [name for name, _ in _GATHERS]
    R = {name: idx_flat[name].shape[0] for name in names}
    per_w = {name: R[name] // NW for name in names}
    idx_off = {}
    off = 0
    for name in names:
        idx_off[name] = off
        off += per_w[name]
    idx_total = off
    out_types = [jax.ShapeDtypeStruct((R[name], D), jnp.float32)
                 for name in names]

    @functools.partial(
        pl.kernel, mesh=mesh,
        out_type=out_types,
        scratch_types=[
            pltpu.VMEM((idx_total,), jnp.int32),
            pltpu.VMEM((CHUNK, D), jnp.float32),
            pltpu.VMEM((CHUNK, D), jnp.float32),
            pltpu.SemaphoreType.DMA,
            pltpu.SemaphoreType.DMA,
            pltpu.SemaphoreType.DMA,
            pltpu.SemaphoreType.DMA,
            pltpu.SemaphoreType.DMA,
        ],
    )
    def k(pat_t, pee_t, ipc_t, ind_t, date_t,
          cemb_i, pemb_i, patn_i, ind_i, fp_i, ipc_i, date_i,
          cemb_o, pemb_o, patn_o, ind_o, fp_o, ipc_o, date_o,
          idx_v, rows0, rows1, isem, gsem0, gsem1, wsem0, wsem1):
        wid = lax.axis_index("s") * 2 + lax.axis_index("c")
        tables = {"cemb": pee_t, "pemb": pat_t, "patn": pat_t, "ind": ind_t,
                  "fp": pee_t, "ipc": ipc_t, "date": date_t}
        idx_in = {"cemb": cemb_i, "pemb": pemb_i, "patn": patn_i,
                  "ind": ind_i, "fp": fp_i, "ipc": ipc_i, "date": date_i}
        outs = {"cemb": cemb_o, "pemb": pemb_o, "patn": patn_o, "ind": ind_o,
                "fp": fp_o, "ipc": ipc_o, "date": date_o}

        pre = []
        for name in names:
            pw = per_w[name]
            pre.append(pltpu.async_copy(
                idx_in[name].at[pl.ds(wid * pw, pw)],
                idx_v.at[pl.ds(idx_off[name], pw)], isem))
        for p in pre:
            p.wait()

        jobs = []
        for name in names:
            pw = per_w[name]
            ch = min(pw, CHUNK)
            for j in range(pw // ch):
                jobs.append((name, j, ch))
        jobs.sort(key=lambda t: -t[2])

        rows = (rows0, rows1)
        gsems = (gsem0, gsem1)
        wsems = (wsem0, wsem1)

        def fire_gather(job, s):
            name, j, ch = job
            src = tables[name].at[
                idx_v.at[pl.ds(idx_off[name] + j * ch, ch)]]
            dst = rows[s] if ch == CHUNK else rows[s].at[pl.ds(0, ch)]
            return pltpu.async_copy(src, dst, gsems[s])

        def fire_wb(job, s):
            name, j, ch = job
            src = rows[s] if ch == CHUNK else rows[s].at[pl.ds(0, ch)]
            dst = outs[name].at[pl.ds(wid * per_w[name] + j * ch, ch)]
            return pltpu.async_copy(src, dst, wsems[s])

        pend_w = [None, None]
        prev = None
        for n, job in enumerate(jobs):
            s = n % 2
            if pend_w[s] is not None:
                pend_w[s].wait()
                pend_w[s] = None
            g = fire_gather(job, s)
            if prev is not None:
                pg, pjob, ps = prev
                pg.wait()
                pend_w[ps] = fire_wb(pjob, ps)
            prev = (g, job, s)
        pg, pjob, ps = prev
        pg.wait()
        pend_w[ps] = fire_wb(pjob, ps)
        for s in (0, 1):
            if pend_w[s] is not None:
                pend_w[s].wait()

    outs = k(patent_table, patentee_table, ipc_table, industry_table,
             appdate_table, *[idx_flat[name] for name in names])
    return {name: o for name, o in zip(names, outs)}


def _compute_body(cemb_ref, pemb_ref, patn_ref, ind_ref, fp_ref, ipc_ref,
                  date_ref, wagg_ref, bagg_ref, wfil_ref, bfil_ref, out_ref):
    wa1 = wagg_ref[:D, :]
    wa2 = wagg_ref[D:, :]
    wf1 = wfil_ref[:D, :]
    wf2 = wfil_ref[D:, :]
    bagg = bagg_ref[...]
    bfil = bfil_ref[...]

    def side(center, groups, n_total):
        bb = center.shape[0]
        c_w = jnp.dot(center, wa1, preferred_element_type=jnp.float32) + bagg
        gsum = jnp.zeros((bb, D), jnp.float32)
        ssum = jnp.zeros((bb, D), jnp.float32)
        for rows, n in groups:
            h = jnp.dot(rows, wa2, preferred_element_type=jnp.float32)
            for k in range(n):
                r = rows[k * bb:(k + 1) * bb, :]
                gate = jax.nn.sigmoid(h[k * bb:(k + 1) * bb, :] + c_w)
                gsum = gsum + r * gate
                ssum = ssum + r
        agg = gsum * (1.0 / n_total)
        avg = ssum * (1.0 / n_total)
        fg = jax.nn.sigmoid(
            jnp.dot(center, wf1, preferred_element_type=jnp.float32)
            + jnp.dot(avg, wf2, preferred_element_type=jnp.float32) + bfil)
        x = center * (1.0 - fg) + agg
        return jnp.where(x >= 0, x, 0.2 * x)

    cemb = cemb_ref[...]
    pemb = pemb_ref[...]
    cs = side(cemb, [(ind_ref[...], 4), (patn_ref[...], 32)], 36.0)
    ps = side(pemb, [(fp_ref[...], 2), (ipc_ref[...], 8), (date_ref[...], 1)],
              11.0)
    out_ref[...] = jax.nn.sigmoid(jnp.sum(cs * ps, axis=1, keepdims=True))


def _tc_compute(cemb, pemb, patn, ind, fp, ipc, date, W_agg, b_agg, W_fil,
                b_fil, bb=TCBB, interpret=False, nrows=B):
    nblk = nrows // bb

    def row_spec(n):
        return pl.BlockSpec((bb * n, D), lambda i: (i, 0))

    full = lambda shape: pl.BlockSpec(shape, lambda i: (0, 0))
    out = pl.pallas_call(
        _compute_body,
        grid=(nblk,),
        in_specs=[
            row_spec(1), row_spec(1), row_spec(32), row_spec(4), row_spec(2),
            row_spec(8), row_spec(1),
            full((2 * D, D)), full((1, D)), full((2 * D, D)), full((1, D)),
        ],
        out_specs=pl.BlockSpec((bb, 1), lambda i: (i, 0)),
        out_shape=jax.ShapeDtypeStruct((nrows, 1), jnp.float32),
        interpret=interpret,
    )(cemb, pemb, patn, ind, fp, ipc, date, W_agg, b_agg.reshape(1, D),
      W_fil, b_fil.reshape(1, D))
    return out.reshape(nrows)


def kernel(company_ids, patent_ids, patent_neighbors, industry_neighbors,
           first_patentee_neighbors, ipc_neighbors, appdate_neighbors,
           patent_table, patentee_table, ipc_table, industry_table,
           appdate_table, W_agg, b_agg, W_fil, b_fil):
    idx_flat = {
        "cemb": company_ids,
        "pemb": patent_ids,
        "patn": patent_neighbors,
        "ind": industry_neighbors,
        "fp": first_patentee_neighbors,
        "ipc": ipc_neighbors,
        "date": appdate_neighbors,
    }
    def perm(a):
        a = a.astype(jnp.int32)
        n = 1 if a.ndim == 1 else a.shape[1]
        return (a.reshape(B // TCBB, TCBB, n).transpose(0, 2, 1).reshape(-1)
                if n > 1 else a.reshape(-1))

    idx1d = {name: perm(a) for name, a in idx_flat.items()}
    S = 4
    bs = B // S
    gathered = []
    for s in range(S):
        sl = {name: lax.slice_in_dim(a, s * (a.shape[0] // S),
                                     (s + 1) * (a.shape[0] // S), axis=0)
              for name, a in idx1d.items()}
        gathered.append(_sc_gather_all(patent_table, patentee_table,
                                       ipc_table, industry_table,
                                       appdate_table, sl))
    outs = []
    for s in range(S):
        rows = gathered[s]
        outs.append(_tc_compute(rows["cemb"], rows["pemb"], rows["patn"],
                                rows["ind"], rows["fp"], rows["ipc"],
                                rows["date"], W_agg, b_agg, W_fil, b_fil,
                                nrows=bs))
    return jnp.concatenate(outs, axis=0)


# speedup vs baseline: 1.6743x; 1.0477x over previous
import functools

import jax
import jax.numpy as jnp
from jax import lax
from jax.experimental import pallas as pl
from jax.experimental.pallas import tpu as pltpu
from jax.experimental.pallas import tpu_sc as plsc

D = 128
B = 4096
CHUNK = 128
NW = 32
TCBB = 256

_GATHERS = (
    ("cemb", 1),
    ("pemb", 1),
    ("patn", 32),
    ("ind", 4),
    ("fp", 2),
    ("ipc", 8),
    ("date", 1),
)


def _sc_gather_all(patent_table, patentee_table, ipc_table, industry_table,
                   appdate_table, idx_flat):
    mesh = plsc.VectorSubcoreMesh(core_axis_name="c", subcore_axis_name="s")
    names = [name for name, _ in _GATHERS]
    R = {name: idx_flat[name].shape[0] for name in names}
    per_w = {name: R[name] // NW for name in names}
    idx_off = {}
    off = 0
    for name in names:
        idx_off[name] = off
        off += per_w[name]
    idx_total = off
    out_types = [jax.ShapeDtypeStruct((R[name], D), jnp.float32)
                 for name in names]

    @functools.partial(
        pl.kernel, mesh=mesh,
        out_type=out_types,
        scratch_types=[
            pltpu.VMEM((idx_total,), jnp.int32),
            pltpu.VMEM((CHUNK, D), jnp.float32),
            pltpu.VMEM((CHUNK, D), jnp.float32),
            pltpu.VMEM((CHUNK, D), jnp.float32),
            pltpu.VMEM((CHUNK, D), jnp.float32),
            pltpu.SemaphoreType.DMA,
            pltpu.SemaphoreType.DMA,
            pltpu.SemaphoreType.DMA,
            pltpu.SemaphoreType.DMA,
            pltpu.SemaphoreType.DMA,
            pltpu.SemaphoreType.DMA,
            pltpu.SemaphoreType.DMA,
            pltpu.SemaphoreType.DMA,
            pltpu.SemaphoreType.DMA,
        ],
    )
    def k(pat_t, pee_t, ipc_t, ind_t, date_t,
          cemb_i, pemb_i, patn_i, ind_i, fp_i, ipc_i, date_i,
          cemb_o, pemb_o, patn_o, ind_o, fp_o, ipc_o, date_o,
          idx_v, rows0, rows1, rows2, rows3, isem,
          gsem0, gsem1, gsem2, gsem3, wsem0, wsem1, wsem2, wsem3):
        wid = lax.axis_index("s") * 2 + lax.axis_index("c")
        tables = {"cemb": pee_t, "pemb": pat_t, "patn": pat_t, "ind": ind_t,
                  "fp": pee_t, "ipc": ipc_t, "date": date_t}
        idx_in = {"cemb": cemb_i, "pemb": pemb_i, "patn": patn_i,
                  "ind": ind_i, "fp": fp_i, "ipc": ipc_i, "date": date_i}
        outs = {"cemb": cemb_o, "pemb": pemb_o, "patn": patn_o, "ind": ind_o,
                "fp": fp_o, "ipc": ipc_o, "date": date_o}

        pre = []
        for name in names:
            pw = per_w[name]
            pre.append(pltpu.async_copy(
                idx_in[name].at[pl.ds(wid * pw, pw)],
                idx_v.at[pl.ds(idx_off[name], pw)], isem))
        for p in pre:
            p.wait()

        jobs = []
        for name in names:
            pw = per_w[name]
            ch = min(pw, CHUNK)
            for j in range(pw // ch):
                jobs.append((name, j, ch))
        jobs.sort(key=lambda t: -t[2])

        rows = (rows0, rows1, rows2, rows3)
        gsems = (gsem0, gsem1, gsem2, gsem3)
        wsems = (wsem0, wsem1, wsem2, wsem3)
        nslot = 4

        def fire_gather(job, s):
            name, j, ch = job
            src = tables[name].at[
                idx_v.at[pl.ds(idx_off[name] + j * ch, ch)]]
            dst = rows[s] if ch == CHUNK else rows[s].at[pl.ds(0, ch)]
            return pltpu.async_copy(src, dst, gsems[s])

        def fire_wb(job, s):
            name, j, ch = job
            src = rows[s] if ch == CHUNK else rows[s].at[pl.ds(0, ch)]
            dst = outs[name].at[pl.ds(wid * per_w[name] + j * ch, ch)]
            return pltpu.async_copy(src, dst, wsems[s])

        pend_w = [None] * nslot
        in_flight = []
        for n, job in enumerate(jobs):
            s = n % nslot
            if pend_w[s] is not None:
                pend_w[s].wait()
                pend_w[s] = None
            in_flight.append((fire_gather(job, s), job, s))
            if len(in_flight) >= nslot - 1:
                pg, pjob, ps = in_flight.pop(0)
                pg.wait()
                pend_w[ps] = fire_wb(pjob, ps)
        for pg, pjob, ps in in_flight:
            pg.wait()
            pend_w[ps] = fire_wb(pjob, ps)
        for s in range(nslot):
            if pend_w[s] is not None:
                pend_w[s].wait()

    outs = k(patent_table, patentee_table, ipc_table, industry_table,
             appdate_table, *[idx_flat[name] for name in names])
    return {name: o for name, o in zip(names, outs)}


def _compute_body(cemb_ref, pemb_ref, patn_ref, ind_ref, fp_ref, ipc_ref,
                  date_ref, wagg_ref, bagg_ref, wfil_ref, bfil_ref, out_ref):
    wa1 = wagg_ref[:D, :]
    wa2 = wagg_ref[D:, :]
    wf1 = wfil_ref[:D, :]
    wf2 = wfil_ref[D:, :]
    bagg = bagg_ref[...]
    bfil = bfil_ref[...]

    def side(center, groups, n_total):
        bb = center.shape[0]
        c_w = jnp.dot(center, wa1, preferred_element_type=jnp.float32) + bagg
        gsum = jnp.zeros((bb, D), jnp.float32)
        ssum = jnp.zeros((bb, D), jnp.float32)
        for rows, n in groups:
            h = jnp.dot(rows, wa2, preferred_element_type=jnp.float32)
            for k in range(n):
                r = rows[k * bb:(k + 1) * bb, :]
                gate = jax.nn.sigmoid(h[k * bb:(k + 1) * bb, :] + c_w)
                gsum = gsum + r * gate
                ssum = ssum + r
        agg = gsum * (1.0 / n_total)
        avg = ssum * (1.0 / n_total)
        fg = jax.nn.sigmoid(
            jnp.dot(center, wf1, preferred_element_type=jnp.float32)
            + jnp.dot(avg, wf2, preferred_element_type=jnp.float32) + bfil)
        x = center * (1.0 - fg) + agg
        return jnp.where(x >= 0, x, 0.2 * x)

    cemb = cemb_ref[...]
    pemb = pemb_ref[...]
    cs = side(cemb, [(ind_ref[...], 4), (patn_ref[...], 32)], 36.0)
    ps = side(pemb, [(fp_ref[...], 2), (ipc_ref[...], 8), (date_ref[...], 1)],
              11.0)
    out_ref[...] = jax.nn.sigmoid(jnp.sum(cs * ps, axis=1, keepdims=True))


def _tc_compute(cemb, pemb, patn, ind, fp, ipc, date, W_agg, b_agg, W_fil,
                b_fil, bb=TCBB, interpret=False, nrows=B):
    nblk = nrows // bb

    def row_spec(n):
        return pl.BlockSpec((bb * n, D), lambda i: (i, 0))

    full = lambda shape: pl.BlockSpec(shape, lambda i: (0, 0))
    out = pl.pallas_call(
        _compute_body,
        grid=(nblk,),
        in_specs=[
            row_spec(1), row_spec(1), row_spec(32), row_spec(4), row_spec(2),
            row_spec(8), row_spec(1),
            full((2 * D, D)), full((1, D)), full((2 * D, D)), full((1, D)),
        ],
        out_specs=pl.BlockSpec((bb, 1), lambda i: (i, 0)),
        out_shape=jax.ShapeDtypeStruct((nrows, 1), jnp.float32),
        interpret=interpret,
    )(cemb, pemb, patn, ind, fp, ipc, date, W_agg, b_agg.reshape(1, D),
      W_fil, b_fil.reshape(1, D))
    return out.reshape(nrows)


def kernel(company_ids, patent_ids, patent_neighbors, industry_neighbors,
           first_patentee_neighbors, ipc_neighbors, appdate_neighbors,
           patent_table, patentee_table, ipc_table, industry_table,
           appdate_table, W_agg, b_agg, W_fil, b_fil):
    idx_flat = {
        "cemb": company_ids,
        "pemb": patent_ids,
        "patn": patent_neighbors,
        "ind": industry_neighbors,
        "fp": first_patentee_neighbors,
        "ipc": ipc_neighbors,
        "date": appdate_neighbors,
    }
    def perm(a):
        a = a.astype(jnp.int32)
        n = 1 if a.ndim == 1 else a.shape[1]
        return (a.reshape(B // TCBB, TCBB, n).transpose(0, 2, 1).reshape(-1)
                if n > 1 else a.reshape(-1))

    idx1d = {name: perm(a) for name, a in idx_flat.items()}
    S = 2
    bs = B // S
    gathered = []
    for s in range(S):
        sl = {name: lax.slice_in_dim(a, s * (a.shape[0] // S),
                                     (s + 1) * (a.shape[0] // S), axis=0)
              for name, a in idx1d.items()}
        gathered.append(_sc_gather_all(patent_table, patentee_table,
                                       ipc_table, industry_table,
                                       appdate_table, sl))
    outs = []
    for s in range(S):
        rows = gathered[s]
        outs.append(_tc_compute(rows["cemb"], rows["pemb"], rows["patn"],
                                rows["ind"], rows["fp"], rows["ipc"],
                                rows["date"], W_agg, b_agg, W_fil, b_fil,
                                nrows=bs))
    return jnp.concatenate(outs, axis=0)
